# baseline (speedup 1.0000x reference)
import jax
import jax.numpy as jnp
from jax import lax
from jax.experimental import pallas as pl
from jax.experimental.pallas import tpu as pltpu
from jax.experimental.pallas import tpu_sc as plsc

B = 16
H = 512
W = 512
N_KP = 2048
LANES = 16
BAND = 128
N_PASS = 2
INV_N = 1.0 / (B * N_KP)
ALPHA = 0.1


def _sc_body(dens_hbm, kx_hbm, ky_hbm, tg_hbm, rgb_hbm, th_hbm, out_hbm,
             kx_v, ky_v, tg_v, band_v, acc_v, red_v, dp_v, shared_v,
             sem, sem_band):
    dens2 = dens_hbm
    c = lax.axis_index("c")
    s = lax.axis_index("s")
    img = s
    ybase = c * (N_PASS * BAND)

    cp_b0 = pltpu.async_copy(dens2.at[pl.ds(img * H + ybase, BAND)],
                             band_v, sem_band)
    cp_kx = pltpu.async_copy(kx_hbm.at[img], kx_v, sem)
    cp_ky = pltpu.async_copy(ky_hbm.at[img], ky_v, sem)
    cp_tg = pltpu.async_copy(tg_hbm.at[img], tg_v, sem)
    cp_kx.wait()
    cp_ky.wait()
    cp_tg.wait()

    def _max_step(j, carry):
        mx, my = carry
        xv = kx_v[pl.ds(j * LANES, LANES)]
        yv = ky_v[pl.ds(j * LANES, LANES)]
        return jnp.maximum(mx, xv), jnp.maximum(my, yv)

    mx0 = kx_v[pl.ds(0, LANES)]
    my0 = ky_v[pl.ds(0, LANES)]
    mx, my = lax.fori_loop(1, N_KP // LANES, _max_step, (mx0, my0))
    max_x = jnp.max(mx)
    max_y = jnp.max(my)

    def _scale(mval, dim):
        mvec = lax.broadcast(mval, (LANES,))
        sc = jnp.full((LANES,), jnp.float32(dim)) / mvec
        return jnp.where(mvec > 0, sc, jnp.full((LANES,), jnp.float32(1.0)))

    scale_w = _scale(max_x, W)
    scale_h = _scale(max_y, H)

    acc = jnp.zeros((LANES,), jnp.float32)
    for p in range(N_PASS):
        y0 = ybase + p * BAND
        if p == 0:
            cp_b0.wait()
        else:
            pltpu.async_copy(dens2.at[pl.ds(img * H + y0, BAND)], band_v,
                             sem_band).wait()

        def _pass_step(j, acc, y0=y0):
            xv = kx_v[pl.ds(j * LANES, LANES)]
            yv = ky_v[pl.ds(j * LANES, LANES)]
            tv = tg_v[pl.ds(j * LANES, LANES)]
            ix = jnp.clip((xv * scale_w).astype(jnp.int32), 0, W - 1)
            iy = jnp.clip((yv * scale_h).astype(jnp.int32), 0, H - 1)
            t = iy - y0
            m = (t >= 0) & (t < BAND)
            pv = plsc.load_gather(band_v, [t, ix], mask=m)
            d = jnp.where(m, pv - tv, jnp.float32(0.0))
            return acc + d * d

        acc = lax.fori_loop(0, N_KP // LANES, _pass_step, acc)

    acc_v[...] = acc
    pltpu.sync_copy(acc_v,
                    shared_v.at[pl.ds((c * LANES + s) * LANES, LANES)])
    plsc.subcore_barrier()

    @pl.when(s == 0)
    def _finalize():
        pltpu.sync_copy(shared_v.at[pl.ds(c * LANES * LANES, LANES * LANES)],
                        red_v)
        tot = red_v[pl.ds(0, LANES)]
        for k in range(1, LANES):
            tot = tot + red_v[pl.ds(k * LANES, LANES)]
        dsum = jnp.sum(tot)

        cp_r = pltpu.async_copy(rgb_hbm, dp_v.at[pl.ds(0, 2 * B)], sem)
        cp_t = pltpu.async_copy(th_hbm, dp_v.at[pl.ds(2 * B, 2 * B)], sem)
        cp_r.wait()
        cp_t.wait()
        iota = lax.broadcasted_iota(jnp.int32, (LANES,), 0)
        even = iota * 2

        def _ce(head, label_col):
            l0 = plsc.load_gather(dp_v, [even + (head * 2 * B)])
            l1 = plsc.load_gather(dp_v, [even + (head * 2 * B + 1)])
            m = jnp.maximum(l0, l1)
            t = jnp.exp(-jnp.abs(l0 - l1))
            y = t
            for _ in range(3):
                y = y - 1.0 + (1.0 + t) * jnp.exp(-y)
            lse = m + y
            picked = l0 if label_col == 0 else l1
            return jnp.sum(lse - picked) * jnp.float32(1.0 / LANES)

        domain = (_ce(0, 0) + _ce(1, 1)) * jnp.float32(0.5)

        res = jnp.where(
            iota == 0, lax.broadcast(dsum, (LANES,)),
            jnp.where(iota == 1, lax.broadcast(domain, (LANES,)),
                      jnp.zeros((LANES,), jnp.float32)))
        acc_v[...] = res
        pltpu.async_copy(acc_v, out_hbm.at[c], sem).wait()


def kernel(density_map, keypoints_list, targets_list,
           domain_pred_rgb, domain_pred_thermal):
    kx = keypoints_list[:, :, 0]
    ky = keypoints_list[:, :, 1]

    mesh = plsc.VectorSubcoreMesh(core_axis_name="c", subcore_axis_name="s")
    sc_kernel = pl.kernel(
        _sc_body,
        out_type=jax.ShapeDtypeStruct((2, LANES), jnp.float32),
        mesh=mesh,
        scratch_types=[
            pltpu.VMEM((N_KP,), jnp.float32),
            pltpu.VMEM((N_KP,), jnp.float32),
            pltpu.VMEM((N_KP,), jnp.float32),
            pltpu.VMEM((BAND, W), jnp.float32),
            pltpu.VMEM((LANES,), jnp.float32),
            pltpu.VMEM((LANES * LANES,), jnp.float32),
            pltpu.VMEM((4 * B,), jnp.float32),
            pltpu.VMEM_SHARED((2 * LANES * LANES,), jnp.float32),
            pltpu.SemaphoreType.DMA,
            pltpu.SemaphoreType.DMA,
        ],
        compiler_params=pltpu.CompilerParams(needs_layout_passes=False),
    )
    p = sc_kernel(density_map.reshape(B * H, W), kx, ky, targets_list,
                  domain_pred_rgb.reshape(2 * B), domain_pred_thermal.reshape(2 * B))

    density_loss = (p[0, 0] + p[1, 0]) * jnp.float32(INV_N)
    domain_loss = p[0, 1]
    total_loss = density_loss + jnp.float32(ALPHA) * domain_loss
    return (total_loss, density_loss, domain_loss)


# speedup vs baseline: 1.1507x; 1.1507x over previous
import jax
import jax.numpy as jnp
from jax import lax
from jax.experimental import pallas as pl
from jax.experimental.pallas import tpu as pltpu
from jax.experimental.pallas import tpu_sc as plsc

B = 16
H = 512
W = 512
N_KP = 2048
LANES = 16
BAND = 120
OWN = 256
N_WORKERS = 32


def _sc_body(dens_hbm, kx_hbm, ky_hbm, tg_hbm, out_hbm,
             kx_v, ky_v, tg_v, band_a, band_b, acc_v, sem, sem_a, sem_b):
    dens2 = dens_hbm
    c = lax.axis_index("c")
    s = lax.axis_index("s")
    wid = s * 2 + c
    img = s
    ybase = c * OWN

    cp_b0 = pltpu.async_copy(dens2.at[pl.ds(img * H + ybase, BAND)],
                             band_a, sem_a)
    cp_b1 = pltpu.async_copy(dens2.at[pl.ds(img * H + ybase + BAND, BAND)],
                             band_b, sem_b)
    cp_kx = pltpu.async_copy(kx_hbm.at[img], kx_v, sem)
    cp_ky = pltpu.async_copy(ky_hbm.at[img], ky_v, sem)
    cp_tg = pltpu.async_copy(tg_hbm.at[img], tg_v, sem)
    cp_kx.wait()
    cp_ky.wait()
    cp_tg.wait()

    def _max_step(j, carry):
        mx, my = carry
        xv = kx_v[pl.ds(j * LANES, LANES)]
        yv = ky_v[pl.ds(j * LANES, LANES)]
        return jnp.maximum(mx, xv), jnp.maximum(my, yv)

    mx0 = kx_v[pl.ds(0, LANES)]
    my0 = ky_v[pl.ds(0, LANES)]
    mx, my = lax.fori_loop(1, N_KP // LANES, _max_step, (mx0, my0))
    max_x = jnp.max(mx)
    max_y = jnp.max(my)

    def _scale(mval, dim):
        mvec = lax.broadcast(mval, (LANES,))
        sc = jnp.full((LANES,), jnp.float32(dim)) / mvec
        return jnp.where(mvec > 0, sc, jnp.full((LANES,), jnp.float32(1.0)))

    scale_w = _scale(max_x, W)
    scale_h = _scale(max_y, H)

    last_rows = OWN - 2 * BAND
    schedule = [(0, BAND, band_a), (BAND, BAND, band_b),
                (2 * BAND, last_rows, band_a)]

    acc = jnp.zeros((LANES,), jnp.float32)
    for p, (off, rows, band) in enumerate(schedule):
        y0 = ybase + off
        if p == 0:
            cp_b0.wait()
        elif p == 1:
            cp_b2 = pltpu.async_copy(
                dens2.at[pl.ds(img * H + ybase + 2 * BAND, last_rows)],
                band_a.at[pl.ds(0, last_rows)], sem_a)
            cp_b1.wait()
        else:
            cp_b2.wait()

        def _pass_step(j, acc, y0=y0, rows=rows, band=band):
            xv = kx_v[pl.ds(j * LANES, LANES)]
            yv = ky_v[pl.ds(j * LANES, LANES)]
            tv = tg_v[pl.ds(j * LANES, LANES)]
            ix = jnp.clip((xv * scale_w).astype(jnp.int32), 0, W - 1)
            iy = jnp.clip((yv * scale_h).astype(jnp.int32), 0, H - 1)
            t = iy - y0
            m = (t >= 0) & (t < rows)
            pv = plsc.load_gather(band, [t, ix], mask=m)
            d = jnp.where(m, pv - tv, jnp.float32(0.0))
            return acc + d * d

        acc = lax.fori_loop(0, N_KP // LANES, _pass_step, acc)

    acc_v[...] = acc
    pltpu.async_copy(acc_v, out_hbm.at[wid], sem).wait()


def _tc_finalize_body(part_ref, rgb_ref, th_ref, out_ref):
    alpha = jnp.float32(0.1)
    density_loss = jnp.sum(part_ref[...]) / jnp.float32(B * N_KP)
    lp_rgb = jax.nn.log_softmax(rgb_ref[...], axis=-1)
    lp_th = jax.nn.log_softmax(th_ref[...], axis=-1)
    ce_rgb = -jnp.mean(lp_rgb[:, 0])
    ce_th = -jnp.mean(lp_th[:, 1])
    domain_loss = (ce_rgb + ce_th) * jnp.float32(0.5)
    out_ref[0] = density_loss + alpha * domain_loss
    out_ref[1] = density_loss
    out_ref[2] = domain_loss


def kernel(density_map, keypoints_list, targets_list,
           domain_pred_rgb, domain_pred_thermal):
    kx = keypoints_list[:, :, 0]
    ky = keypoints_list[:, :, 1]

    mesh = plsc.VectorSubcoreMesh(core_axis_name="c", subcore_axis_name="s")
    sc_kernel = pl.kernel(
        _sc_body,
        out_type=jax.ShapeDtypeStruct((N_WORKERS, LANES), jnp.float32),
        mesh=mesh,
        scratch_types=[
            pltpu.VMEM((N_KP,), jnp.float32),
            pltpu.VMEM((N_KP,), jnp.float32),
            pltpu.VMEM((N_KP,), jnp.float32),
            pltpu.VMEM((BAND, W), jnp.float32),
            pltpu.VMEM((BAND, W), jnp.float32),
            pltpu.VMEM((LANES,), jnp.float32),
            pltpu.SemaphoreType.DMA,
            pltpu.SemaphoreType.DMA,
            pltpu.SemaphoreType.DMA,
        ],
        compiler_params=pltpu.CompilerParams(needs_layout_passes=False),
    )
    partials = sc_kernel(density_map.reshape(B * H, W), kx, ky, targets_list)

    out = pl.pallas_call(
        _tc_finalize_body,
        out_shape=jax.ShapeDtypeStruct((3,), jnp.float32),
        out_specs=pl.BlockSpec(memory_space=pltpu.SMEM),
    )(partials, domain_pred_rgb, domain_pred_thermal)

    return (out[0], out[1], out[2])


# speedup vs baseline: 1.2265x; 1.0659x over previous
import jax
import jax.numpy as jnp
from jax import lax
from jax.experimental import pallas as pl
from jax.experimental.pallas import tpu as pltpu
from jax.experimental.pallas import tpu_sc as plsc

B = 16
H = 512
W = 512
N_KP = 2048
LANES = 16
BAND = 128
OWN = 256
N_WORKERS = 32


def _sc_body(dens_hbm, kx_hbm, ky_hbm, tg_hbm, out_hbm,
             kx_v, ky_v, tg_v, band_a, acc_v, sem, sem_a):
    dens2 = dens_hbm
    c = lax.axis_index("c")
    s = lax.axis_index("s")
    wid = s * 2 + c
    img = s
    ybase = c * OWN

    cp_b0 = pltpu.async_copy(dens2.at[pl.ds(img * H + ybase, BAND)],
                             band_a, sem_a)
    cp_kx = pltpu.async_copy(kx_hbm.at[img], kx_v, sem)
    cp_ky = pltpu.async_copy(ky_hbm.at[img], ky_v, sem)
    cp_tg = pltpu.async_copy(tg_hbm.at[img], tg_v, sem)
    cp_kx.wait()
    cp_ky.wait()
    cp_tg.wait()

    def _max_step(j, carry):
        mx, my = carry
        xv = kx_v[pl.ds(j * LANES, LANES)]
        yv = ky_v[pl.ds(j * LANES, LANES)]
        return jnp.maximum(mx, xv), jnp.maximum(my, yv)

    mx0 = kx_v[pl.ds(0, LANES)]
    my0 = ky_v[pl.ds(0, LANES)]
    mx, my = lax.fori_loop(1, N_KP // LANES, _max_step, (mx0, my0))
    max_x = jnp.max(mx)
    max_y = jnp.max(my)

    def _scale(mval, dim):
        mvec = lax.broadcast(mval, (LANES,))
        sc = jnp.full((LANES,), jnp.float32(dim)) / mvec
        return jnp.where(mvec > 0, sc, jnp.full((LANES,), jnp.float32(1.0)))

    scale_w = _scale(max_x, W)
    scale_h = _scale(max_y, H)

    acc = jnp.zeros((LANES,), jnp.float32)
    for p in range(2):
        y0 = ybase + p * BAND
        rows = BAND
        band = band_a
        if p == 0:
            cp_b0.wait()
        else:
            pltpu.async_copy(dens2.at[pl.ds(img * H + y0, BAND)], band_a,
                             sem_a).wait()

        def _pass_step(j, acc, y0=y0, rows=rows, band=band):
            xv = kx_v[pl.ds(j * LANES, LANES)]
            yv = ky_v[pl.ds(j * LANES, LANES)]
            tv = tg_v[pl.ds(j * LANES, LANES)]
            ix = jnp.clip((xv * scale_w).astype(jnp.int32), 0, W - 1)
            iy = jnp.clip((yv * scale_h).astype(jnp.int32), 0, H - 1)
            t = iy - y0
            m = (t >= 0) & (t < rows)
            pv = plsc.load_gather(band, [t, ix], mask=m)
            d = jnp.where(m, pv - tv, jnp.float32(0.0))
            return acc + d * d

        acc = lax.fori_loop(0, N_KP // LANES, _pass_step, acc)

    acc_v[...] = acc
    pltpu.async_copy(acc_v, out_hbm.at[wid], sem).wait()


def _tc_finalize_body(part_ref, rgb_ref, th_ref, tot_ref, den_ref, dom_ref):
    alpha = jnp.float32(0.1)
    density_loss = jnp.sum(part_ref[...]) / jnp.float32(B * N_KP)
    lp_rgb = jax.nn.log_softmax(rgb_ref[...], axis=-1)
    lp_th = jax.nn.log_softmax(th_ref[...], axis=-1)
    ce_rgb = -jnp.mean(lp_rgb[:, 0])
    ce_th = -jnp.mean(lp_th[:, 1])
    domain_loss = (ce_rgb + ce_th) * jnp.float32(0.5)
    tot_ref[0] = density_loss + alpha * domain_loss
    den_ref[0] = density_loss
    dom_ref[0] = domain_loss


def kernel(density_map, keypoints_list, targets_list,
           domain_pred_rgb, domain_pred_thermal):
    kx = keypoints_list[:, :, 0]
    ky = keypoints_list[:, :, 1]

    mesh = plsc.VectorSubcoreMesh(core_axis_name="c", subcore_axis_name="s")
    sc_kernel = pl.kernel(
        _sc_body,
        out_type=jax.ShapeDtypeStruct((N_WORKERS, LANES), jnp.float32),
        mesh=mesh,
        scratch_types=[
            pltpu.VMEM((N_KP,), jnp.float32),
            pltpu.VMEM((N_KP,), jnp.float32),
            pltpu.VMEM((N_KP,), jnp.float32),
            pltpu.VMEM((BAND, W), jnp.float32),
            pltpu.VMEM((LANES,), jnp.float32),
            pltpu.SemaphoreType.DMA,
            pltpu.SemaphoreType.DMA,
        ],
        compiler_params=pltpu.CompilerParams(needs_layout_passes=False),
    )
    partials = sc_kernel(density_map.reshape(B * H, W), kx, ky, targets_list)

    tot, den, dom = pl.pallas_call(
        _tc_finalize_body,
        out_shape=[jax.ShapeDtypeStruct((1,), jnp.float32)] * 3,
        out_specs=[pl.BlockSpec(memory_space=pltpu.SMEM)] * 3,
    )(partials, domain_pred_rgb, domain_pred_thermal)

    return (tot.reshape(()), den.reshape(()), dom.reshape(()))
